# baseline (speedup 1.0000x reference)
import functools

import jax
import jax.numpy as jnp
from jax import lax
from jax.experimental import pallas as pl
from jax.experimental.pallas import tpu as pltpu
from jax.experimental.pallas import tpu_sc as plsc

B, N, D = 64, 1024, 512
NC, NS, L = 2, 16, 16
NW = NC * NS
CHUNK = (N // NW) * D
VECS = CHUNK // L
TOKV = D // L


def _sc_body(x_hbm, pe_hbm, tok_hbm, out_hbm, pe_v, tok_v, buf):
    wid = lax.axis_index("s") * NC + lax.axis_index("c")
    base = wid * CHUNK

    pltpu.sync_copy(pe_hbm.at[pl.ds(base, CHUNK)], pe_v)
    pltpu.sync_copy(tok_hbm, tok_v)

    def _pretok(k, _):
        ds = pl.ds(k * L, L)
        pe_v[ds] = pe_v[ds] + tok_v[pl.ds(lax.rem(k, TOKV) * L, L)]
        return 0
    lax.fori_loop(0, VECS, _pretok, 0)

    def _img(b, _):
        off = b * (N * D) + base
        pltpu.sync_copy(x_hbm.at[pl.ds(off, CHUNK)], buf)

        def _add(k, _):
            ds = pl.ds(k * L, L)
            buf[ds] = buf[ds] + pe_v[ds]
            return 0
        lax.fori_loop(0, VECS, _add, 0)
        pltpu.sync_copy(buf, out_hbm.at[pl.ds(off, CHUNK)])
        return 0
    lax.fori_loop(0, B, _img, 0)


def kernel(x, enc_mask, pos_embed, mask_token, embed_token):
    n_patches = enc_mask.shape[1]
    n_masked = n_patches - x.shape[1]

    sc_add = pl.kernel(
        _sc_body,
        out_type=jax.ShapeDtypeStruct((B * N * D,), x.dtype),
        mesh=plsc.VectorSubcoreMesh(core_axis_name="c", subcore_axis_name="s"),
        scratch_types=[
            pltpu.VMEM((CHUNK,), jnp.float32),
            pltpu.VMEM((D,), jnp.float32),
            pltpu.VMEM((CHUNK,), jnp.float32),
        ],
    )
    x_vis = sc_add(
        x.reshape(-1), pos_embed.reshape(-1), embed_token.reshape(-1)
    ).reshape(B, N, D)

    x_mask = jnp.zeros((B, n_masked, D), x.dtype)
    return (x_vis, x_mask)


# speedup vs baseline: 8.2377x; 8.2377x over previous
import jax
import jax.numpy as jnp
from jax.experimental import pallas as pl
from jax.experimental.pallas import tpu as pltpu


def _add_pe_kernel(x_ref, pe_ref, tok_ref, out_ref):
    out_ref[...] = x_ref[...] + (pe_ref[...] + tok_ref[...])[None]


def kernel(x, enc_mask, pos_embed, mask_token, embed_token):
    B, N, D = x.shape
    n_patches = enc_mask.shape[1]
    n_masked = n_patches - N
    tok = embed_token.reshape(1, D)

    BB = 4
    x_vis = pl.pallas_call(
        _add_pe_kernel,
        grid=(B // BB,),
        in_specs=[
            pl.BlockSpec((BB, N, D), lambda b: (b, 0, 0)),
            pl.BlockSpec((N, D), lambda b: (0, 0)),
            pl.BlockSpec((1, D), lambda b: (0, 0)),
        ],
        out_specs=pl.BlockSpec((BB, N, D), lambda b: (b, 0, 0)),
        out_shape=jax.ShapeDtypeStruct((B, N, D), x.dtype),
        compiler_params=pltpu.CompilerParams(
            dimension_semantics=("parallel",),
        ),
    )(x, pos_embed, tok)

    x_mask = jnp.zeros((B, n_masked, D), x.dtype)
    return (x_vis, x_mask)
